# Initial kernel scaffold; baseline (speedup 1.0000x reference)
import jax
import jax.numpy as jnp
from jax.experimental import pallas as pl


def kernel(x, target):
    raise NotImplementedError("write your pallas kernel here")


# speedup vs baseline: 1.8651x; 1.8651x over previous
import math

import jax
import jax.numpy as jnp
from jax.experimental import pallas as pl
from jax.experimental.pallas import tpu as pltpu

_SIZE = 100000
_N = 2048
_PAD = 0
_SMOOTH = 0.1
_EPS = _SMOOTH / (_SIZE - 2)
_CONF = 1.0 - _SMOOTH
_K = (_SIZE - 2) * _EPS * math.log(_EPS) + _CONF * math.log(_CONF)

_BLK = 2048
_GRID = (_SIZE + _BLK - 1) // _BLK


def _body(x_ref, t_ref, out_ref, acc_ref):
    c = pl.program_id(0)

    @pl.when(c == 0)
    def _init():
        acc_ref[...] = jnp.zeros_like(acc_ref)

    xblk = x_ref[...]
    cols = c * _BLK + jax.lax.broadcasted_iota(jnp.int32, xblk.shape, 1)
    xm = jnp.where(cols < _SIZE, xblk, 0.0)
    t = t_ref[...]
    psum = jnp.sum(xm, axis=1, keepdims=True)
    gsel = jnp.sum(jnp.where(cols == t, xm, 0.0), axis=1, keepdims=True)
    part = (-_EPS) * psum + (_EPS - _CONF) * gsel

    @pl.when(c == 0)
    def _col0():
        acc_ref[...] = acc_ref[...] + _EPS * xblk[:, 0:1]

    acc_ref[...] = acc_ref[...] + part

    @pl.when(c == _GRID - 1)
    def _final():
        m = (t != _PAD).astype(jnp.float32)
        out_ref[...] = jnp.sum(m * (_K + acc_ref[...]), keepdims=True)


def kernel(x, target):
    t2d = target.astype(jnp.int32).reshape(_N, 1)
    out = pl.pallas_call(
        _body,
        grid=(_GRID,),
        in_specs=[
            pl.BlockSpec((_N, _BLK), lambda c: (0, c)),
            pl.BlockSpec((_N, 1), lambda c: (0, 0)),
        ],
        out_specs=pl.BlockSpec((1, 1), lambda c: (0, 0)),
        out_shape=jax.ShapeDtypeStruct((1, 1), jnp.float32),
        scratch_shapes=[pltpu.VMEM((_N, 1), jnp.float32)],
    )(x, t2d)
    return out.reshape(())
